# baseline (speedup 1.0000x reference)
import functools

import jax
import jax.numpy as jnp
from jax import lax
from jax.experimental import pallas as pl
from jax.experimental.pallas import tpu as pltpu
from jax.experimental.pallas import tpu_sc as plsc

N = 10000
E = 320000
D = 128

NC = 2
NS = 16
NW = NC * NS
EPT = E // NW
CH = 80
NCHUNK = EPT // CH
RA = 624
TAIL = N - NS * RA
ZSRC = 80
DGW = 16

_MESH = plsc.VectorSubcoreMesh(core_axis_name="c", subcore_axis_name="s")


def _zero_acc(sid, zero_v, acc_sh):
    base = pl.multiple_of(sid * RA, 8)
    off = 0
    while off < RA:
        step = min(ZSRC, RA - off)
        pltpu.sync_copy(zero_v.at[pl.ds(0, step)], acc_sh.at[pl.ds(base + off, step)])
        off += step

    @pl.when(sid == NS - 1)
    def _():
        pltpu.sync_copy(zero_v.at[pl.ds(0, TAIL)], acc_sh.at[pl.ds(NS * RA, TAIL)])


def _write_out(sid, cid, acc_sh, out_hbm):
    base = pl.multiple_of(sid * RA, 8)
    pltpu.sync_copy(acc_sh.at[pl.ds(base, RA)], out_hbm.at[cid, pl.ds(base, RA)])

    @pl.when(sid == NS - 1)
    def _():
        pltpu.sync_copy(acc_sh.at[pl.ds(NS * RA, TAIL)],
                        out_hbm.at[cid, pl.ds(NS * RA, TAIL)])


@functools.partial(
    pl.kernel,
    out_type=(
        jax.ShapeDtypeStruct((NC, N, DGW), jnp.float32),
        jax.ShapeDtypeStruct((NC, N, DGW), jnp.float32),
    ),
    mesh=_MESH,
    scratch_types=[
        pltpu.VMEM((2, NCHUNK, CH), jnp.int32),
        pltpu.VMEM((CH, DGW), jnp.float32),
        pltpu.VMEM((ZSRC, DGW), jnp.float32),
        pltpu.VMEM_SHARED((N, DGW), jnp.float32),
        pltpu.VMEM_SHARED((N, DGW), jnp.float32),
    ],
)
def _deg_kernel(e3, dego_hbm, degi_hbm, idx_v, ones_v, zero_v, dego_sh, degi_sh):
    cid = lax.axis_index("c")
    sid = lax.axis_index("s")
    wid = sid * NC + cid

    pltpu.sync_copy(e3.at[0, wid], idx_v.at[0])
    pltpu.sync_copy(e3.at[1, wid], idx_v.at[1])

    def fill(i, carry):
        ones_v[i] = jnp.ones((DGW,), jnp.float32)
        return carry

    lax.fori_loop(0, CH, fill, 0)

    def zfill(i, carry):
        zero_v[i] = jnp.zeros((DGW,), jnp.float32)
        return carry

    lax.fori_loop(0, ZSRC, zfill, 0)

    _zero_acc(sid, zero_v, dego_sh)
    _zero_acc(sid, zero_v, degi_sh)
    plsc.subcore_barrier()

    def chunk(j, carry):
        pltpu.sync_copy(ones_v, dego_sh.at[idx_v.at[0, j]], add=True)
        pltpu.sync_copy(ones_v, degi_sh.at[idx_v.at[1, j]], add=True)
        return carry

    lax.fori_loop(0, NCHUNK, chunk, 0)
    plsc.subcore_barrier()

    _write_out(sid, cid, dego_sh, dego_hbm)
    _write_out(sid, cid, degi_sh, degi_hbm)


NB = 3

@functools.partial(
    pl.kernel,
    out_type=jax.ShapeDtypeStruct((NC, N, D), jnp.float32),
    mesh=_MESH,
    scratch_types=[
        pltpu.VMEM((EPT,), jnp.int32),
        pltpu.VMEM((NB, 1, CH), jnp.int32),
        pltpu.VMEM((NB, CH, D), jnp.float32),
        pltpu.VMEM_SHARED((N, D), jnp.float32),
        [pltpu.SemaphoreType.DMA] * NB,
        [pltpu.SemaphoreType.DMA] * NB,
        [pltpu.SemaphoreType.DMA] * NB,
    ],
)
def _agg_kernel(e2, e3d, xs_hbm, out_hbm, idxs_v, stage_v, rows_v,
                acc_sh, gsem, ssem, tsem):
    cid = lax.axis_index("c")
    sid = lax.axis_index("s")
    wid = sid * NC + cid

    pltpu.sync_copy(e2.at[0, wid], idxs_v)

    def gidx(j):
        return idxs_v.at[pl.ds(pl.multiple_of(j * CH, 8), CH)]

    def fire_gather(j, b):
        pltpu.async_copy(xs_hbm.at[gidx(j)], rows_v.at[b], gsem[b])

    def wait_gather(j, b):
        pltpu.make_async_copy(xs_hbm.at[gidx(j)], rows_v.at[b], gsem[b]).wait()

    def fire_scatter(j, b):
        pltpu.async_copy(rows_v.at[b], acc_sh.at[stage_v.at[b, 0]], ssem[b],
                         add=True)

    def wait_scatter(j, b):
        pltpu.make_async_copy(rows_v.at[b], acc_sh.at[stage_v.at[b, 0]],
                              ssem[b]).wait()

    def fire_stage(j, b):
        pltpu.async_copy(e3d.at[wid, j], stage_v.at[b], tsem[b])

    def wait_stage(j, b):
        pltpu.make_async_copy(e3d.at[wid, j], stage_v.at[b], tsem[b]).wait()

    for j in range(1, NB):
        fire_gather(j, j)

    def zfill(i, carry):
        rows_v[0, i // 8, pl.ds((i % 8) * 16, 16)] = jnp.zeros((16,), jnp.float32)
        return carry

    lax.fori_loop(0, CH * (D // 16), zfill, 0)

    _zero_acc(sid, rows_v.at[0], acc_sh)
    fire_gather(0, 0)
    for j in range(NB):
        pltpu.sync_copy(e3d.at[wid, j], stage_v.at[j])
    plsc.subcore_barrier()

    def step(j, b, swait, gfire, stwait):
        wait_gather(j, b)
        if stwait:
            wait_stage(j, b)
        fire_scatter(j, b)
        if swait:
            wait_scatter(j - 2, (b - 2) % NB)
        if gfire:
            fire_gather(j + NB - 2, (b - 2) % NB)
            fire_stage(j + NB - 2, (b - 2) % NB)

    GLAST = NCHUNK + 1 - NB
    PEEL = max(2, NB)
    while PEEL % NB:
        PEEL += 1
    for j in range(PEEL):
        step(j, j % NB, swait=(j >= 2), gfire=(j >= 2 and j <= GLAST),
             stwait=(j >= NB))
    MAIN = (min(GLAST, NCHUNK - 1) - PEEL + 1) // NB

    def chunk(k, carry):
        for u in range(NB):
            step(NB * k + u + PEEL, (u + PEEL) % NB, swait=True, gfire=True,
                 stwait=True)
        return carry

    lax.fori_loop(0, MAIN, chunk, 0)

    for j in range(PEEL + NB * MAIN, NCHUNK):
        step(j, j % NB, swait=True, gfire=(j <= GLAST), stwait=(j >= NB))
    for j in range(NCHUNK - 2, NCHUNK):
        wait_scatter(j, j % NB)
    plsc.subcore_barrier()

    _write_out(sid, cid, acc_sh, out_hbm)


HR = 80
HC = 128
HN = HR * HC

@functools.partial(
    pl.kernel,
    out_type=(
        jax.ShapeDtypeStruct((NW, HN), jnp.float32),
        jax.ShapeDtypeStruct((NW, HN), jnp.float32),
    ),
    mesh=_MESH,
    scratch_types=[
        pltpu.VMEM((2, EPT), jnp.int32),
        pltpu.VMEM((HN,), jnp.float32),
        pltpu.VMEM((HN,), jnp.float32),
    ],
    compiler_params=pltpu.CompilerParams(needs_layout_passes=False),
)
def _hist_kernel(e2, dego_hbm, degi_hbm, idx_v, h0, h1):
    cid = lax.axis_index("c")
    sid = lax.axis_index("s")
    wid = sid * NC + cid

    pltpu.sync_copy(e2.at[0, wid], idx_v.at[0])
    pltpu.sync_copy(e2.at[1, wid], idx_v.at[1])

    zero16 = jnp.zeros((16,), jnp.float32)

    def zfill(i, carry):
        off = pl.multiple_of(i * 16, 16)
        h0[pl.ds(off, 16)] = zero16
        h1[pl.ds(off, 16)] = zero16
        return carry

    lax.fori_loop(0, HN // 16, zfill, 0)

    ones16 = jnp.ones((16,), jnp.float32)

    def acc(i, carry):
        off = pl.multiple_of(i * 16, 16)
        plsc.addupdate_scatter(h0, [idx_v[0, pl.ds(off, 16)]], ones16)
        plsc.addupdate_scatter(h1, [idx_v[1, pl.ds(off, 16)]], ones16)
        return carry

    lax.fori_loop(0, EPT // 16, acc, 0)

    pltpu.sync_copy(h0, dego_hbm.at[wid])
    pltpu.sync_copy(h1, degi_hbm.at[wid])


def _norm_body(d0_ref, d1_ref, no_ref, ni_ref):
    dgo = jnp.sum(d0_ref[...], axis=0, keepdims=True)
    no_ref[...] = lax.rsqrt(jnp.maximum(dgo, 1.0))
    dgi = jnp.sum(d1_ref[...], axis=0, keepdims=True)
    ni_ref[...] = lax.rsqrt(jnp.maximum(dgi, 1.0))


_HB = 1024


def _norm_call(degp_o, degp_i):
    return pl.pallas_call(
        _norm_body,
        grid=(HN // _HB,),
        in_specs=[
            pl.BlockSpec((NW, _HB), lambda i: (0, i)),
            pl.BlockSpec((NW, _HB), lambda i: (0, i)),
        ],
        out_specs=[
            pl.BlockSpec((1, _HB), lambda i: (0, i)),
            pl.BlockSpec((1, _HB), lambda i: (0, i)),
        ],
        out_shape=[
            jax.ShapeDtypeStruct((1, HN), jnp.float32),
            jax.ShapeDtypeStruct((1, HN), jnp.float32),
        ],
    )(degp_o, degp_i)


_R = 1000


def _prep_body(x_ref, nout_ref, xs_ref):
    xs_ref[...] = x_ref[...] * nout_ref[...]


def _prep_call(x, nout_c):
    return pl.pallas_call(
        _prep_body,
        grid=(N // _R,),
        in_specs=[
            pl.BlockSpec((_R, D), lambda i: (i, 0)),
            pl.BlockSpec((_R, 1), lambda i: (i, 0)),
        ],
        out_specs=pl.BlockSpec((_R, D), lambda i: (i, 0)),
        out_shape=jax.ShapeDtypeStruct((N, D), jnp.float32),
    )(x, nout_c)


def _mm_body(aggp_ref, nin_ref, nout_ref, w_ref, b_ref, out_ref, *, act):
    agg = aggp_ref[0] + aggp_ref[1]
    h = jnp.dot(agg, w_ref[...], preferred_element_type=jnp.float32)
    h = h * nin_ref[...] + b_ref[...]
    if act:
        h = jnp.maximum(h, 0.0) * nout_ref[...]
    out_ref[...] = h


def _mm_call(aggp, nin_c, nout_c, w, b, act):
    return pl.pallas_call(
        functools.partial(_mm_body, act=act),
        grid=(N // _R,),
        in_specs=[
            pl.BlockSpec((NC, _R, D), lambda i: (0, i, 0)),
            pl.BlockSpec((_R, 1), lambda i: (i, 0)),
            pl.BlockSpec((_R, 1), lambda i: (i, 0)),
            pl.BlockSpec((D, D), lambda i: (0, 0)),
            pl.BlockSpec((1, D), lambda i: (0, 0)),
        ],
        out_specs=pl.BlockSpec((_R, D), lambda i: (i, 0)),
        out_shape=jax.ShapeDtypeStruct((N, D), jnp.float32),
    )(aggp, nin_c, nout_c, w, b)


@jax.jit
def kernel(x, edge_index, W1, b1, W2, b2):
    e2 = edge_index.reshape(2, NW, EPT)
    e3d = edge_index[1].reshape(NW, NCHUNK, 1, CH)
    degp_o, degp_i = _hist_kernel(e2)
    no_mat, ni_mat = _norm_call(degp_o, degp_i)
    nout_c = no_mat.reshape(HN)[:N].reshape(N, 1)
    nin_c = ni_mat.reshape(HN)[:N].reshape(N, 1)
    xs = _prep_call(x, nout_c)
    aggp1 = _agg_kernel(e2, e3d, xs)
    hs = _mm_call(aggp1, nin_c, nout_c, W1, b1.reshape(1, D), act=True)
    aggp2 = _agg_kernel(e2, e3d, hs)
    return _mm_call(aggp2, nin_c, nout_c, W2, b2.reshape(1, D), act=False)


# speedup vs baseline: 1.2531x; 1.2531x over previous
import functools

import jax
import jax.numpy as jnp
from jax import lax
from jax.experimental import pallas as pl
from jax.experimental.pallas import tpu as pltpu
from jax.experimental.pallas import tpu_sc as plsc

N = 10000
E = 320000
D = 128

NC = 2
NS = 16
NW = NC * NS
EPT = E // NW
CH = 80
NCHUNK = EPT // CH
RA = 624
TAIL = N - NS * RA
ZSRC = 80
DGW = 16

_MESH = plsc.VectorSubcoreMesh(core_axis_name="c", subcore_axis_name="s")


def _zero_acc(sid, zero_v, acc_sh):
    base = pl.multiple_of(sid * RA, 8)
    off = 0
    while off < RA:
        step = min(ZSRC, RA - off)
        pltpu.sync_copy(zero_v.at[pl.ds(0, step)], acc_sh.at[pl.ds(base + off, step)])
        off += step

    @pl.when(sid == NS - 1)
    def _():
        pltpu.sync_copy(zero_v.at[pl.ds(0, TAIL)], acc_sh.at[pl.ds(NS * RA, TAIL)])


def _write_out(sid, cid, acc_sh, out_hbm):
    base = pl.multiple_of(sid * RA, 8)
    pltpu.sync_copy(acc_sh.at[pl.ds(base, RA)], out_hbm.at[cid, pl.ds(base, RA)])

    @pl.when(sid == NS - 1)
    def _():
        pltpu.sync_copy(acc_sh.at[pl.ds(NS * RA, TAIL)],
                        out_hbm.at[cid, pl.ds(NS * RA, TAIL)])


@functools.partial(
    pl.kernel,
    out_type=(
        jax.ShapeDtypeStruct((NC, N, DGW), jnp.float32),
        jax.ShapeDtypeStruct((NC, N, DGW), jnp.float32),
    ),
    mesh=_MESH,
    scratch_types=[
        pltpu.VMEM((2, NCHUNK, CH), jnp.int32),
        pltpu.VMEM((CH, DGW), jnp.float32),
        pltpu.VMEM((ZSRC, DGW), jnp.float32),
        pltpu.VMEM_SHARED((N, DGW), jnp.float32),
        pltpu.VMEM_SHARED((N, DGW), jnp.float32),
    ],
)
def _deg_kernel(e3, dego_hbm, degi_hbm, idx_v, ones_v, zero_v, dego_sh, degi_sh):
    cid = lax.axis_index("c")
    sid = lax.axis_index("s")
    wid = sid * NC + cid

    pltpu.sync_copy(e3.at[0, wid], idx_v.at[0])
    pltpu.sync_copy(e3.at[1, wid], idx_v.at[1])

    def fill(i, carry):
        ones_v[i] = jnp.ones((DGW,), jnp.float32)
        return carry

    lax.fori_loop(0, CH, fill, 0)

    def zfill(i, carry):
        zero_v[i] = jnp.zeros((DGW,), jnp.float32)
        return carry

    lax.fori_loop(0, ZSRC, zfill, 0)

    _zero_acc(sid, zero_v, dego_sh)
    _zero_acc(sid, zero_v, degi_sh)
    plsc.subcore_barrier()

    def chunk(j, carry):
        pltpu.sync_copy(ones_v, dego_sh.at[idx_v.at[0, j]], add=True)
        pltpu.sync_copy(ones_v, degi_sh.at[idx_v.at[1, j]], add=True)
        return carry

    lax.fori_loop(0, NCHUNK, chunk, 0)
    plsc.subcore_barrier()

    _write_out(sid, cid, dego_sh, dego_hbm)
    _write_out(sid, cid, degi_sh, degi_hbm)


@functools.partial(
    pl.kernel,
    out_type=jax.ShapeDtypeStruct((NC, N, D), jnp.float32),
    mesh=_MESH,
    scratch_types=[
        pltpu.VMEM((EPT,), jnp.int32),
        pltpu.VMEM((NCHUNK, CH), jnp.int32),
        pltpu.VMEM((2, CH, D), jnp.float32),
        pltpu.VMEM_SHARED((N, D), jnp.float32),
        pltpu.SemaphoreType.DMA,
        pltpu.SemaphoreType.DMA,
        pltpu.SemaphoreType.DMA,
    ],
)
def _agg_kernel(e2, e3d, xs_hbm, out_hbm, idxs_v, idxd_v, rows_v, acc_sh,
                sem0, sem1, semi):
    cid = lax.axis_index("c")
    sid = lax.axis_index("s")
    wid = sid * NC + cid

    pltpu.sync_copy(e2.at[0, wid], idxs_v)
    pltpu.async_copy(e3d.at[wid], idxd_v, semi)

    def gidx(j):
        return idxs_v.at[pl.ds(pl.multiple_of(j * CH, 8), CH)]

    sems = (sem0, sem1)
    pltpu.async_copy(xs_hbm.at[gidx(1)], rows_v.at[1], sem1)

    def zfill(i, carry):
        rows_v[0, i // 8, pl.ds((i % 8) * 16, 16)] = jnp.zeros((16,), jnp.float32)
        return carry

    lax.fori_loop(0, CH * (D // 16), zfill, 0)

    _zero_acc(sid, rows_v.at[0], acc_sh)
    pltpu.async_copy(xs_hbm.at[gidx(0)], rows_v.at[0], sem0)
    pltpu.make_async_copy(e3d.at[wid], idxd_v, semi).wait()
    plsc.subcore_barrier()

    main_iters = (NCHUNK - 3) // 2

    def chunk(k, carry):
        for b in range(2):
            j = 2 * k + b
            pltpu.make_async_copy(xs_hbm.at[gidx(j)], rows_v.at[b], sems[b]).wait()
            pltpu.sync_copy(rows_v.at[b], acc_sh.at[idxd_v.at[j]], add=True)
            pltpu.async_copy(xs_hbm.at[gidx(j + 2)], rows_v.at[b], sems[b])
        return carry

    lax.fori_loop(0, main_iters, chunk, 0)

    for j in range(2 * main_iters, NCHUNK):
        b = j % 2
        pltpu.make_async_copy(xs_hbm.at[gidx(j)], rows_v.at[b], sems[b]).wait()
        pltpu.sync_copy(rows_v.at[b], acc_sh.at[idxd_v.at[j]], add=True)
        if j + 2 < NCHUNK:
            pltpu.async_copy(xs_hbm.at[gidx(j + 2)], rows_v.at[b], sems[b])
    plsc.subcore_barrier()

    _write_out(sid, cid, acc_sh, out_hbm)


HR = 80
HC = 128
HN = HR * HC

@functools.partial(
    pl.kernel,
    out_type=(
        jax.ShapeDtypeStruct((NW, HN), jnp.float32),
        jax.ShapeDtypeStruct((NW, HN), jnp.float32),
    ),
    mesh=_MESH,
    scratch_types=[
        pltpu.VMEM((2, EPT), jnp.int32),
        pltpu.VMEM((HN,), jnp.float32),
        pltpu.VMEM((HN,), jnp.float32),
    ],
    compiler_params=pltpu.CompilerParams(needs_layout_passes=False),
)
def _hist_kernel(e2, dego_hbm, degi_hbm, idx_v, h0, h1):
    cid = lax.axis_index("c")
    sid = lax.axis_index("s")
    wid = sid * NC + cid

    pltpu.sync_copy(e2.at[0, wid], idx_v.at[0])
    pltpu.sync_copy(e2.at[1, wid], idx_v.at[1])

    zero16 = jnp.zeros((16,), jnp.float32)

    def zfill(i, carry):
        off = pl.multiple_of(i * 16, 16)
        h0[pl.ds(off, 16)] = zero16
        h1[pl.ds(off, 16)] = zero16
        return carry

    lax.fori_loop(0, HN // 16, zfill, 0)

    ones16 = jnp.ones((16,), jnp.float32)

    def acc(i, carry):
        off = pl.multiple_of(i * 16, 16)
        plsc.addupdate_scatter(h0, [idx_v[0, pl.ds(off, 16)]], ones16)
        plsc.addupdate_scatter(h1, [idx_v[1, pl.ds(off, 16)]], ones16)
        return carry

    lax.fori_loop(0, EPT // 16, acc, 0)

    pltpu.sync_copy(h0, dego_hbm.at[wid])
    pltpu.sync_copy(h1, degi_hbm.at[wid])


def _norm_body(d0_ref, d1_ref, no_ref, ni_ref):
    dgo = jnp.sum(d0_ref[...], axis=0, keepdims=True)
    no_ref[...] = lax.rsqrt(jnp.maximum(dgo, 1.0))
    dgi = jnp.sum(d1_ref[...], axis=0, keepdims=True)
    ni_ref[...] = lax.rsqrt(jnp.maximum(dgi, 1.0))


_HB = 1024


def _norm_call(degp_o, degp_i):
    return pl.pallas_call(
        _norm_body,
        grid=(HN // _HB,),
        in_specs=[
            pl.BlockSpec((NW, _HB), lambda i: (0, i)),
            pl.BlockSpec((NW, _HB), lambda i: (0, i)),
        ],
        out_specs=[
            pl.BlockSpec((1, _HB), lambda i: (0, i)),
            pl.BlockSpec((1, _HB), lambda i: (0, i)),
        ],
        out_shape=[
            jax.ShapeDtypeStruct((1, HN), jnp.float32),
            jax.ShapeDtypeStruct((1, HN), jnp.float32),
        ],
    )(degp_o, degp_i)


_R = 1000


def _prep_body(x_ref, nout_ref, xs_ref):
    xs_ref[...] = x_ref[...] * nout_ref[...]


def _prep_call(x, nout_c):
    return pl.pallas_call(
        _prep_body,
        grid=(N // _R,),
        in_specs=[
            pl.BlockSpec((_R, D), lambda i: (i, 0)),
            pl.BlockSpec((_R, 1), lambda i: (i, 0)),
        ],
        out_specs=pl.BlockSpec((_R, D), lambda i: (i, 0)),
        out_shape=jax.ShapeDtypeStruct((N, D), jnp.float32),
    )(x, nout_c)


def _mm_body(aggp_ref, nin_ref, nout_ref, w_ref, b_ref, out_ref, *, act):
    agg = aggp_ref[0] + aggp_ref[1]
    h = jnp.dot(agg, w_ref[...], preferred_element_type=jnp.float32)
    h = h * nin_ref[...] + b_ref[...]
    if act:
        h = jnp.maximum(h, 0.0) * nout_ref[...]
    out_ref[...] = h


def _mm_call(aggp, nin_c, nout_c, w, b, act):
    return pl.pallas_call(
        functools.partial(_mm_body, act=act),
        grid=(N // _R,),
        in_specs=[
            pl.BlockSpec((NC, _R, D), lambda i: (0, i, 0)),
            pl.BlockSpec((_R, 1), lambda i: (i, 0)),
            pl.BlockSpec((_R, 1), lambda i: (i, 0)),
            pl.BlockSpec((D, D), lambda i: (0, 0)),
            pl.BlockSpec((1, D), lambda i: (0, 0)),
        ],
        out_specs=pl.BlockSpec((_R, D), lambda i: (i, 0)),
        out_shape=jax.ShapeDtypeStruct((N, D), jnp.float32),
    )(aggp, nin_c, nout_c, w, b)


@jax.jit
def kernel(x, edge_index, W1, b1, W2, b2):
    e2 = edge_index.reshape(2, NW, EPT)
    e3d = edge_index[1].reshape(NW, NCHUNK, CH)
    degp_o, degp_i = _hist_kernel(e2)
    no_mat, ni_mat = _norm_call(degp_o, degp_i)
    nout_c = no_mat.reshape(HN)[:N].reshape(N, 1)
    nin_c = ni_mat.reshape(HN)[:N].reshape(N, 1)
    xs = _prep_call(x, nout_c)
    aggp1 = _agg_kernel(e2, e3d, xs)
    hs = _mm_call(aggp1, nin_c, nout_c, W1, b1.reshape(1, D), act=True)
    aggp2 = _agg_kernel(e2, e3d, hs)
    return _mm_call(aggp2, nin_c, nout_c, W2, b2.reshape(1, D), act=False)
